# baseline (speedup 1.0000x reference)
import jax
import jax.numpy as jnp
from jax import lax
from jax.experimental import pallas as pl
from jax.experimental.pallas import tpu as pltpu
from jax.experimental.pallas import tpu_sc as plsc

N = 16777216
NUM_BINS = 10
NBND = NUM_BINS - 1

NC = 2
NS = 16
LANES = 16
NW = NC * NS
SC_SHARE = 9
N_SC = SC_SHARE * (N // 32)
PER_TILE = N_SC // NW
CHUNK = 8192
NCHUNK = PER_TILE // CHUNK
NVEC = CHUNK // LANES

N_TC = N - N_SC
BR = 1024
ROWS = N // 128
OFF_BLK = N_SC // (BR * 128)
G = N_TC // (BR * 128)
NQ = 2 * NBND + 1


def _sc_tile_body(conf_hbm, acc_hbm, bnds_hbm, out_hbm,
                  cbuf, abuf, bbuf, obuf,
                  sem_c0, sem_c1, sem_a0, sem_a1, sem_b, sem_o):
  cid = lax.axis_index("c")
  sid = lax.axis_index("s")
  wid = sid * NC + cid
  base = wid * PER_TILE

  pltpu.async_copy(bnds_hbm, bbuf, sem_b).wait()
  bvecs = [bbuf[i, :] for i in range(NBND)]

  sems_c = (sem_c0, sem_c1)
  sems_a = (sem_a0, sem_a1)

  def start(k, b):
    pltpu.async_copy(conf_hbm.at[pl.ds(base + k * CHUNK, CHUNK)],
                     cbuf.at[b], sems_c[b])
    pltpu.async_copy(acc_hbm.at[pl.ds(base + k * CHUNK, CHUNK)],
                     abuf.at[b], sems_a[b])

  start(0, 0)
  start(1, 1)

  zrow = jnp.zeros((LANES,), jnp.float32)
  zrow_i = jnp.zeros((LANES,), jnp.int32)
  init = (zrow,) * (1 + NBND) + (zrow_i,) * NBND

  def process(b, carry):
    def vbody(j, acc):
      off = j * LANES
      c = cbuf[b, pl.ds(off, LANES)]
      a = abuf[b, pl.ds(off, LANES)]
      s = list(acc)
      s[0] = s[0] + a
      for i in range(NBND):
        m = c >= bvecs[i]
        s[1 + i] = s[1 + i] + jnp.where(m, a, jnp.float32(0.0))
        s[1 + NBND + i] = s[1 + NBND + i] + plsc.all_reduce_population_count(m)
      return tuple(s)
    return lax.fori_loop(0, NVEC, vbody, carry, unroll=1)

  def outer(k2, carry):
    for b in range(2):
      k = 2 * k2 + b
      pltpu.make_async_copy(conf_hbm.at[pl.ds(base + k * CHUNK, CHUNK)],
                            cbuf.at[b], sems_c[b]).wait()
      pltpu.make_async_copy(acc_hbm.at[pl.ds(base + k * CHUNK, CHUNK)],
                            abuf.at[b], sems_a[b]).wait()
      carry = process(b, carry)

      @pl.when(k + 2 < NCHUNK)
      def _():
        start(k + 2, b)
    return carry

  accs = lax.fori_loop(0, NCHUNK // 2, outer, init)

  lane = lax.iota(jnp.int32, LANES)

  def lane_sum(x):
    for k in (1, 2, 4, 8):
      x = x + x.at[jnp.bitwise_xor(lane, k)].get(mode="promise_in_bounds")
    return x

  zrow_f = zrow
  S = [lane_sum(accs[i]) for i in range(NUM_BINS)] + [zrow_f]
  C = ([jnp.full((LANES,), float(PER_TILE), jnp.float32)]
       + [accs[1 + NBND + i].astype(jnp.float32) for i in range(NBND)]
       + [zrow_f])
  cnt_row = zrow_f
  acc_row = zrow_f
  for i in range(NUM_BINS):
    cnt_row = jnp.where(lane == i, C[i] - C[i + 1], cnt_row)
    acc_row = jnp.where(lane == i, S[i] - S[i + 1], acc_row)
  obuf[0, :] = cnt_row
  obuf[1, :] = acc_row
  pltpu.async_copy(obuf, out_hbm.at[wid], sem_o).wait()


def _tc_body(bnds_ref, c_ref, a_ref, out_ref, acc_ref):
  step = pl.program_id(0)

  @pl.when(step == 0)
  def _():
    acc_ref[...] = jnp.zeros_like(acc_ref)

  bvecs = [bnds_ref[i, :] for i in range(NBND)]

  def sub(r, carry):
    c = c_ref[pl.ds(r * 8, 8), :]
    a = a_ref[pl.ds(r * 8, 8), :]
    s = list(carry)
    s[0] = s[0] + a
    for i in range(NBND):
      m = c >= bvecs[i]
      s[1 + i] = s[1 + i] + jnp.where(m, a, 0.0)
      s[1 + NBND + i] = s[1 + NBND + i] + jnp.where(m, 1.0, 0.0)
    return tuple(s)

  zero = jnp.zeros((8, 128), jnp.float32)
  accs = (zero,) * NQ
  for r in range(BR // 8):
    accs = sub(r, accs)
  for q in range(NQ):
    acc_ref[q] = acc_ref[q] + accs[q]

  @pl.when(step == G - 1)
  def _():
    S = [jnp.sum(acc_ref[i]) for i in range(NUM_BINS)] + [jnp.float32(0.0)]
    C = ([jnp.float32(float(N_TC))]
         + [jnp.sum(acc_ref[1 + NBND + i]) for i in range(NBND)]
         + [jnp.float32(0.0)])
    sub_i = lax.broadcasted_iota(jnp.int32, (8, 128), 0)
    lane_i = lax.broadcasted_iota(jnp.int32, (8, 128), 1)
    mat = jnp.zeros((8, 128), jnp.float32)
    for i in range(NUM_BINS):
      mat = jnp.where((sub_i == 0) & (lane_i == i), C[i] - C[i + 1], mat)
      mat = jnp.where((sub_i == 1) & (lane_i == i), S[i] - S[i + 1], mat)
    out_ref[...] = mat


def kernel(confidences, accuracies):
  boundaries = jnp.linspace(0.0, 1.0, NUM_BINS + 1, dtype=jnp.float32)
  bnds_sc = jnp.asarray(
      jnp.broadcast_to(boundaries[1:NUM_BINS, None], (NBND, LANES)),
      dtype=jnp.float32)
  bnds_tc = jnp.asarray(
      jnp.broadcast_to(boundaries[1:NUM_BINS, None], (NBND, 128)),
      dtype=jnp.float32)

  run_sc = pl.kernel(
      _sc_tile_body,
      out_type=jax.ShapeDtypeStruct((NW, 2, LANES), jnp.float32),
      mesh=plsc.VectorSubcoreMesh(core_axis_name="c", subcore_axis_name="s"),
      compiler_params=pltpu.CompilerParams(
          use_tc_tiling_on_sc=False, needs_layout_passes=False),
      scratch_types=[
          pltpu.VMEM((2, CHUNK), jnp.float32),
          pltpu.VMEM((2, CHUNK), jnp.float32),
          pltpu.VMEM((NBND, LANES), jnp.float32),
          pltpu.VMEM((2, LANES), jnp.float32),
          pltpu.SemaphoreType.DMA,
          pltpu.SemaphoreType.DMA,
          pltpu.SemaphoreType.DMA,
          pltpu.SemaphoreType.DMA,
          pltpu.SemaphoreType.DMA,
          pltpu.SemaphoreType.DMA,
      ],
  )
  sc_partials = run_sc(confidences, accuracies, bnds_sc)

  c2 = confidences.reshape(ROWS, 128)
  a2 = accuracies.reshape(ROWS, 128)
  tc_out = pl.pallas_call(
      _tc_body,
      grid=(G,),
      in_specs=[
          pl.BlockSpec((NBND, 128), lambda g: (0, 0)),
          pl.BlockSpec((BR, 128), lambda g: (g + OFF_BLK, 0)),
          pl.BlockSpec((BR, 128), lambda g: (g + OFF_BLK, 0)),
      ],
      out_specs=pl.BlockSpec((8, 128), lambda g: (0, 0)),
      out_shape=jax.ShapeDtypeStruct((8, 128), jnp.float32),
      scratch_shapes=[pltpu.VMEM((NQ, 8, 128), jnp.float32)],
      compiler_params=pltpu.CompilerParams(
          dimension_semantics=("arbitrary",)),
  )(bnds_tc, c2, a2)

  return sc_partials.sum(axis=0)[:, :NUM_BINS] + tc_out[:2, :NUM_BINS]


# speedup vs baseline: 1.0335x; 1.0335x over previous
import jax
import jax.numpy as jnp
from jax import lax
from jax.experimental import pallas as pl
from jax.experimental.pallas import tpu as pltpu
from jax.experimental.pallas import tpu_sc as plsc

N = 16777216
NUM_BINS = 10
NBND = NUM_BINS - 1

NC = 2
NS = 16
LANES = 16
NW = NC * NS
SC_SHARE = 10
N_SC = SC_SHARE * (N // 32)
PER_TILE = N_SC // NW
CHUNK = 8192
NCHUNK = PER_TILE // CHUNK
NVEC = CHUNK // LANES

N_TC = N - N_SC
BR = 1024
ROWS = N // 128
OFF_BLK = N_SC // (BR * 128)
G = N_TC // (BR * 128)
NQ = 2 * NBND + 1


def _sc_tile_body(conf_hbm, acc_hbm, bnds_hbm, out_hbm,
                  cbuf, abuf, bbuf, obuf,
                  sem_c0, sem_c1, sem_a0, sem_a1, sem_b, sem_o):
  cid = lax.axis_index("c")
  sid = lax.axis_index("s")
  wid = sid * NC + cid
  base = wid * PER_TILE

  pltpu.async_copy(bnds_hbm, bbuf, sem_b).wait()
  bvecs = [bbuf[i, :] for i in range(NBND)]

  sems_c = (sem_c0, sem_c1)
  sems_a = (sem_a0, sem_a1)

  def start(k, b):
    pltpu.async_copy(conf_hbm.at[pl.ds(base + k * CHUNK, CHUNK)],
                     cbuf.at[b], sems_c[b])
    pltpu.async_copy(acc_hbm.at[pl.ds(base + k * CHUNK, CHUNK)],
                     abuf.at[b], sems_a[b])

  start(0, 0)
  start(1, 1)

  zrow = jnp.zeros((LANES,), jnp.float32)
  zrow_i = jnp.zeros((LANES,), jnp.int32)
  init = (zrow,) * (1 + NBND) + (zrow_i,) * NBND

  def process(b, carry):
    def vbody(j, acc):
      off = j * LANES
      c = cbuf[b, pl.ds(off, LANES)]
      a = abuf[b, pl.ds(off, LANES)]
      s = list(acc)
      s[0] = s[0] + a
      for i in range(NBND):
        m = c >= bvecs[i]
        s[1 + i] = s[1 + i] + jnp.where(m, a, jnp.float32(0.0))
        s[1 + NBND + i] = s[1 + NBND + i] + plsc.all_reduce_population_count(m)
      return tuple(s)
    return lax.fori_loop(0, NVEC, vbody, carry, unroll=1)

  def outer(k2, carry):
    for b in range(2):
      k = 2 * k2 + b
      pltpu.make_async_copy(conf_hbm.at[pl.ds(base + k * CHUNK, CHUNK)],
                            cbuf.at[b], sems_c[b]).wait()
      pltpu.make_async_copy(acc_hbm.at[pl.ds(base + k * CHUNK, CHUNK)],
                            abuf.at[b], sems_a[b]).wait()
      carry = process(b, carry)

      @pl.when(k + 2 < NCHUNK)
      def _():
        start(k + 2, b)
    return carry

  accs = lax.fori_loop(0, NCHUNK // 2, outer, init)

  lane = lax.iota(jnp.int32, LANES)

  def lane_sum(x):
    for k in (1, 2, 4, 8):
      x = x + x.at[jnp.bitwise_xor(lane, k)].get(mode="promise_in_bounds")
    return x

  zrow_f = zrow
  S = [lane_sum(accs[i]) for i in range(NUM_BINS)] + [zrow_f]
  C = ([jnp.full((LANES,), float(PER_TILE), jnp.float32)]
       + [accs[1 + NBND + i].astype(jnp.float32) for i in range(NBND)]
       + [zrow_f])
  cnt_row = zrow_f
  acc_row = zrow_f
  for i in range(NUM_BINS):
    cnt_row = jnp.where(lane == i, C[i] - C[i + 1], cnt_row)
    acc_row = jnp.where(lane == i, S[i] - S[i + 1], acc_row)
  obuf[0, :] = cnt_row
  obuf[1, :] = acc_row
  pltpu.async_copy(obuf, out_hbm.at[wid], sem_o).wait()


def _tc_body(bnds_ref, c_ref, a_ref, out_ref, acc_ref):
  step = pl.program_id(0)

  @pl.when(step == 0)
  def _():
    acc_ref[...] = jnp.zeros_like(acc_ref)

  bvecs = [jnp.broadcast_to(bnds_ref[i, :], (8, 128)) + jnp.zeros((8, 128), jnp.float32)
           for i in range(NBND)]

  def sub(r, carry):
    c = c_ref[pl.ds(r * 8, 8), :]
    a = a_ref[pl.ds(r * 8, 8), :]
    s = list(carry)
    s[0] = s[0] + a
    for i in range(NBND):
      m = c >= bvecs[i]
      s[1 + i] = s[1 + i] + jnp.where(m, a, 0.0)
      s[1 + NBND + i] = s[1 + NBND + i] + jnp.where(m, 1.0, 0.0)
    return tuple(s)

  zero = jnp.zeros((8, 128), jnp.float32)
  accs = (zero,) * NQ
  for r in range(BR // 8):
    accs = sub(r, accs)
  for q in range(NQ):
    acc_ref[q] = acc_ref[q] + accs[q]

  @pl.when(step == G - 1)
  def _():
    S = [jnp.sum(acc_ref[i]) for i in range(NUM_BINS)] + [jnp.float32(0.0)]
    C = ([jnp.float32(float(N_TC))]
         + [jnp.sum(acc_ref[1 + NBND + i]) for i in range(NBND)]
         + [jnp.float32(0.0)])
    sub_i = lax.broadcasted_iota(jnp.int32, (8, 128), 0)
    lane_i = lax.broadcasted_iota(jnp.int32, (8, 128), 1)
    mat = jnp.zeros((8, 128), jnp.float32)
    for i in range(NUM_BINS):
      mat = jnp.where((sub_i == 0) & (lane_i == i), C[i] - C[i + 1], mat)
      mat = jnp.where((sub_i == 1) & (lane_i == i), S[i] - S[i + 1], mat)
    out_ref[...] = mat


def kernel(confidences, accuracies):
  boundaries = jnp.linspace(0.0, 1.0, NUM_BINS + 1, dtype=jnp.float32)
  bnds_sc = jnp.asarray(
      jnp.broadcast_to(boundaries[1:NUM_BINS, None], (NBND, LANES)),
      dtype=jnp.float32)
  bnds_tc = jnp.asarray(
      jnp.broadcast_to(boundaries[1:NUM_BINS, None], (NBND, 128)),
      dtype=jnp.float32)

  run_sc = pl.kernel(
      _sc_tile_body,
      out_type=jax.ShapeDtypeStruct((NW, 2, LANES), jnp.float32),
      mesh=plsc.VectorSubcoreMesh(core_axis_name="c", subcore_axis_name="s"),
      compiler_params=pltpu.CompilerParams(
          use_tc_tiling_on_sc=False, needs_layout_passes=False),
      scratch_types=[
          pltpu.VMEM((2, CHUNK), jnp.float32),
          pltpu.VMEM((2, CHUNK), jnp.float32),
          pltpu.VMEM((NBND, LANES), jnp.float32),
          pltpu.VMEM((2, LANES), jnp.float32),
          pltpu.SemaphoreType.DMA,
          pltpu.SemaphoreType.DMA,
          pltpu.SemaphoreType.DMA,
          pltpu.SemaphoreType.DMA,
          pltpu.SemaphoreType.DMA,
          pltpu.SemaphoreType.DMA,
      ],
  )
  sc_partials = run_sc(confidences, accuracies, bnds_sc)

  c2 = confidences.reshape(ROWS, 128)
  a2 = accuracies.reshape(ROWS, 128)
  tc_out = pl.pallas_call(
      _tc_body,
      grid=(G,),
      in_specs=[
          pl.BlockSpec((NBND, 128), lambda g: (0, 0)),
          pl.BlockSpec((BR, 128), lambda g: (g + OFF_BLK, 0)),
          pl.BlockSpec((BR, 128), lambda g: (g + OFF_BLK, 0)),
      ],
      out_specs=pl.BlockSpec((8, 128), lambda g: (0, 0)),
      out_shape=jax.ShapeDtypeStruct((8, 128), jnp.float32),
      scratch_shapes=[pltpu.VMEM((NQ, 8, 128), jnp.float32)],
      compiler_params=pltpu.CompilerParams(
          dimension_semantics=("arbitrary",)),
  )(bnds_tc, c2, a2)

  return sc_partials.sum(axis=0)[:, :NUM_BINS] + tc_out[:2, :NUM_BINS]


# speedup vs baseline: 1.0374x; 1.0038x over previous
import jax
import jax.numpy as jnp
from jax import lax
from jax.experimental import pallas as pl
from jax.experimental.pallas import tpu as pltpu
from jax.experimental.pallas import tpu_sc as plsc

N = 16777216
NUM_BINS = 10
NBND = NUM_BINS - 1

NC = 2
NS = 16
LANES = 16
NW = NC * NS
SC_SHARE = 10
N_SC = SC_SHARE * (N // 32)
PER_TILE = N_SC // NW
CHUNK = 8192
NCHUNK = PER_TILE // CHUNK
NVEC = CHUNK // LANES

N_TC = N - N_SC
BR = 1024
ROWS = N // 128
OFF_BLK = N_SC // (BR * 128)
G = N_TC // (BR * 128)
NQ = 2 * NBND + 1


def _sc_tile_body(conf_hbm, acc_hbm, bnds_hbm, out_hbm,
                  cbuf, abuf, bbuf, obuf,
                  sem_c0, sem_c1, sem_a0, sem_a1, sem_b, sem_o):
  cid = lax.axis_index("c")
  sid = lax.axis_index("s")
  wid = sid * NC + cid
  base = wid * PER_TILE

  pltpu.async_copy(bnds_hbm, bbuf, sem_b).wait()
  bvecs = [bbuf[i, :] for i in range(NBND)]

  sems_c = (sem_c0, sem_c1)
  sems_a = (sem_a0, sem_a1)

  def start(k, b):
    pltpu.async_copy(conf_hbm.at[pl.ds(base + k * CHUNK, CHUNK)],
                     cbuf.at[b], sems_c[b])
    pltpu.async_copy(acc_hbm.at[pl.ds(base + k * CHUNK, CHUNK)],
                     abuf.at[b], sems_a[b])

  start(0, 0)
  start(1, 1)

  zrow = jnp.zeros((LANES,), jnp.float32)
  zrow_i = jnp.zeros((LANES,), jnp.int32)
  init = (zrow,) * (1 + NBND) + (zrow_i,) * NBND

  def process(b, carry):
    def vbody(j, acc):
      off = j * LANES
      c = cbuf[b, pl.ds(off, LANES)]
      a = abuf[b, pl.ds(off, LANES)]
      s = list(acc)
      s[0] = s[0] + a
      for i in range(NBND):
        m = c >= bvecs[i]
        s[1 + i] = s[1 + i] + jnp.where(m, a, jnp.float32(0.0))
        s[1 + NBND + i] = s[1 + NBND + i] + plsc.all_reduce_population_count(m)
      return tuple(s)
    return lax.fori_loop(0, NVEC, vbody, carry, unroll=1)

  def outer(k2, carry):
    for b in range(2):
      k = 2 * k2 + b
      pltpu.make_async_copy(conf_hbm.at[pl.ds(base + k * CHUNK, CHUNK)],
                            cbuf.at[b], sems_c[b]).wait()
      pltpu.make_async_copy(acc_hbm.at[pl.ds(base + k * CHUNK, CHUNK)],
                            abuf.at[b], sems_a[b]).wait()
      carry = process(b, carry)

      @pl.when(k + 2 < NCHUNK)
      def _():
        start(k + 2, b)
    return carry

  accs = lax.fori_loop(0, NCHUNK // 2, outer, init)

  lane = lax.iota(jnp.int32, LANES)

  def lane_sum(x):
    for k in (1, 2, 4, 8):
      x = x + x.at[jnp.bitwise_xor(lane, k)].get(mode="promise_in_bounds")
    return x

  zrow_f = zrow
  S = [lane_sum(accs[i]) for i in range(NUM_BINS)] + [zrow_f]
  C = ([jnp.full((LANES,), float(PER_TILE), jnp.float32)]
       + [accs[1 + NBND + i].astype(jnp.float32) for i in range(NBND)]
       + [zrow_f])
  cnt_row = zrow_f
  acc_row = zrow_f
  for i in range(NUM_BINS):
    cnt_row = jnp.where(lane == i, C[i] - C[i + 1], cnt_row)
    acc_row = jnp.where(lane == i, S[i] - S[i + 1], acc_row)
  obuf[0, :] = cnt_row
  obuf[1, :] = acc_row
  pltpu.async_copy(obuf, out_hbm.at[wid], sem_o).wait()


def _tc_body(bnds_ref, c_ref, a_ref, out_ref, acc_ref):
  step = pl.program_id(0)

  @pl.when(step == 0)
  def _():
    acc_ref[...] = jnp.zeros_like(acc_ref)

  bvecs = [jnp.broadcast_to(bnds_ref[i, :], (8, 128)) + jnp.zeros((8, 128), jnp.float32)
           for i in range(NBND)]

  def sub(r, carry):
    c = c_ref[pl.ds(r * 8, 8), :]
    a = a_ref[pl.ds(r * 8, 8), :]
    s = list(carry)
    s[0] = s[0] + a
    for i in range(NBND):
      mf = jnp.where(c >= bvecs[i], 1.0, 0.0)
      s[1 + i] = s[1 + i] + mf * a
      s[1 + NBND + i] = s[1 + NBND + i] + mf
    return tuple(s)

  zero = jnp.zeros((8, 128), jnp.float32)
  accs = (zero,) * NQ
  for r in range(BR // 8):
    accs = sub(r, accs)
  for q in range(NQ):
    acc_ref[q] = acc_ref[q] + accs[q]

  @pl.when(step == G - 1)
  def _():
    S = [jnp.sum(acc_ref[i]) for i in range(NUM_BINS)] + [jnp.float32(0.0)]
    C = ([jnp.float32(float(N_TC))]
         + [jnp.sum(acc_ref[1 + NBND + i]) for i in range(NBND)]
         + [jnp.float32(0.0)])
    sub_i = lax.broadcasted_iota(jnp.int32, (8, 128), 0)
    lane_i = lax.broadcasted_iota(jnp.int32, (8, 128), 1)
    mat = jnp.zeros((8, 128), jnp.float32)
    for i in range(NUM_BINS):
      mat = jnp.where((sub_i == 0) & (lane_i == i), C[i] - C[i + 1], mat)
      mat = jnp.where((sub_i == 1) & (lane_i == i), S[i] - S[i + 1], mat)
    out_ref[...] = mat


def kernel(confidences, accuracies):
  boundaries = jnp.linspace(0.0, 1.0, NUM_BINS + 1, dtype=jnp.float32)
  bnds_sc = jnp.asarray(
      jnp.broadcast_to(boundaries[1:NUM_BINS, None], (NBND, LANES)),
      dtype=jnp.float32)
  bnds_tc = jnp.asarray(
      jnp.broadcast_to(boundaries[1:NUM_BINS, None], (NBND, 128)),
      dtype=jnp.float32)

  run_sc = pl.kernel(
      _sc_tile_body,
      out_type=jax.ShapeDtypeStruct((NW, 2, LANES), jnp.float32),
      mesh=plsc.VectorSubcoreMesh(core_axis_name="c", subcore_axis_name="s"),
      compiler_params=pltpu.CompilerParams(
          use_tc_tiling_on_sc=False, needs_layout_passes=False),
      scratch_types=[
          pltpu.VMEM((2, CHUNK), jnp.float32),
          pltpu.VMEM((2, CHUNK), jnp.float32),
          pltpu.VMEM((NBND, LANES), jnp.float32),
          pltpu.VMEM((2, LANES), jnp.float32),
          pltpu.SemaphoreType.DMA,
          pltpu.SemaphoreType.DMA,
          pltpu.SemaphoreType.DMA,
          pltpu.SemaphoreType.DMA,
          pltpu.SemaphoreType.DMA,
          pltpu.SemaphoreType.DMA,
      ],
  )
  sc_partials = run_sc(confidences, accuracies, bnds_sc)

  c2 = confidences.reshape(ROWS, 128)
  a2 = accuracies.reshape(ROWS, 128)
  tc_out = pl.pallas_call(
      _tc_body,
      grid=(G,),
      in_specs=[
          pl.BlockSpec((NBND, 128), lambda g: (0, 0)),
          pl.BlockSpec((BR, 128), lambda g: (g + OFF_BLK, 0)),
          pl.BlockSpec((BR, 128), lambda g: (g + OFF_BLK, 0)),
      ],
      out_specs=pl.BlockSpec((8, 128), lambda g: (0, 0)),
      out_shape=jax.ShapeDtypeStruct((8, 128), jnp.float32),
      scratch_shapes=[pltpu.VMEM((NQ, 8, 128), jnp.float32)],
      compiler_params=pltpu.CompilerParams(
          dimension_semantics=("arbitrary",)),
  )(bnds_tc, c2, a2)

  return sc_partials.sum(axis=0)[:, :NUM_BINS] + tc_out[:2, :NUM_BINS]
